# baseline (speedup 1.0000x reference)
import jax
import jax.numpy as jnp
from jax.experimental import pallas as pl
from jax.experimental.pallas import tpu as pltpu

PROTO_MOMENTUM = 0.001


def _update_kernel(feats_ref, act_ref, vid_ref, proto_ref, out_ref,
                   sum_ref, cnt_ref):
    j = pl.program_id(1)
    nb = pl.num_programs(1)

    @pl.when(j == 0)
    def _init():
        sum_ref[...] = jnp.zeros_like(sum_ref)
        cnt_ref[...] = jnp.zeros_like(cnt_ref)

    G, C, T = act_ref.shape
    acc = sum_ref[...]
    cacc = cnt_ref[...]
    for g in range(G):
        vid_row = vid_ref[pl.ds(j * G + g, 1), :]
        vid_col = jnp.transpose(vid_row, (1, 0))
        mask = (act_ref[g] != 0) & (vid_col != 0)
        maskb = mask.astype(jnp.bfloat16)
        featsb = feats_ref[g].astype(jnp.bfloat16)
        cacc = cacc + jax.lax.dot_general(
            maskb, jnp.ones((T, 128), jnp.bfloat16),
            dimension_numbers=(((1,), (0,)), ((), ())),
            preferred_element_type=jnp.float32)
        acc = acc + jax.lax.dot_general(
            maskb, featsb,
            dimension_numbers=(((1,), (0,)), ((), ())),
            preferred_element_type=jnp.float32)
    sum_ref[...] = acc
    cnt_ref[...] = cacc

    @pl.when(j == nb - 1)
    def _finish():
        cnt = cnt_ref[:, 0:1]
        mean = sum_ref[...] * (1.0 / jnp.maximum(cnt, 1.0))
        proto = proto_ref[:, 0, :]
        upd = (1.0 - PROTO_MOMENTUM) * proto + PROTO_MOMENTUM * mean
        out_ref[:, 0, :] = jnp.where(cnt > 0.0, upd, proto)


def kernel(feats, act_seq, vid_label, proto_vectors):
    B, T, D = feats.shape
    C = act_seq.shape[2]
    P = proto_vectors.shape[1]
    act_t = jnp.swapaxes(act_seq, 1, 2)

    G = 2
    H = 2
    Dh = D // H
    return pl.pallas_call(
        _update_kernel,
        grid=(H, B // G),
        in_specs=[
            pl.BlockSpec((G, T, Dh), lambda h, j: (j, 0, h)),
            pl.BlockSpec((G, C, T), lambda h, j: (j, 0, 0)),
            pl.BlockSpec((B, C), lambda h, j: (0, 0)),
            pl.BlockSpec((C, P, Dh), lambda h, j: (0, 0, h)),
        ],
        out_specs=pl.BlockSpec((C, P, Dh), lambda h, j: (0, 0, h)),
        out_shape=jax.ShapeDtypeStruct((C, P, D), jnp.float32),
        scratch_shapes=[
            pltpu.VMEM((C, Dh), jnp.float32),
            pltpu.VMEM((C, 128), jnp.float32),
        ],
        compiler_params=pltpu.CompilerParams(
            dimension_semantics=("parallel", "arbitrary")),
    )(feats, act_t, vid_label, proto_vectors)


# speedup vs baseline: 1.2163x; 1.2163x over previous
import jax
import jax.numpy as jnp
from jax.experimental import pallas as pl
from jax.experimental.pallas import tpu as pltpu

PROTO_MOMENTUM = 0.001


def _update_kernel(feats_ref, act_ref, vid_ref, proto_ref, out_ref,
                   sum_ref, cnt_ref):
    b = pl.program_id(0)
    nb = pl.num_programs(0)

    @pl.when(b == 0)
    def _init():
        sum_ref[...] = jnp.zeros_like(sum_ref)
        cnt_ref[...] = jnp.zeros_like(cnt_ref)

    G, C, T = act_ref.shape
    acc = sum_ref[...]
    masks = []
    for g in range(G):
        vid_row = vid_ref[pl.ds(b * G + g, 1), :]
        vid_col = jnp.transpose(vid_row, (1, 0))
        mask = (act_ref[g] != 0) & (vid_col != 0)
        maskb = mask.astype(jnp.bfloat16)
        masks.append(maskb)
        featsb = feats_ref[g].astype(jnp.bfloat16)
        acc = acc + jax.lax.dot_general(
            maskb, featsb,
            dimension_numbers=(((1,), (0,)), ((), ())),
            preferred_element_type=jnp.float32)
    sum_ref[...] = acc
    cnt_ref[...] += jax.lax.dot_general(
        sum(masks[1:], masks[0]), jnp.ones((T, 128), jnp.bfloat16),
        dimension_numbers=(((1,), (0,)), ((), ())),
        preferred_element_type=jnp.float32)

    @pl.when(b == nb - 1)
    def _finish():
        cnt = cnt_ref[:, 0:1]
        mean = sum_ref[...] * (1.0 / jnp.maximum(cnt, 1.0))
        proto = proto_ref[:, 0, :]
        upd = (1.0 - PROTO_MOMENTUM) * proto + PROTO_MOMENTUM * mean
        out_ref[:, 0, :] = jnp.where(cnt > 0.0, upd, proto)


def kernel(feats, act_seq, vid_label, proto_vectors):
    B, T, D = feats.shape
    C = act_seq.shape[2]
    P = proto_vectors.shape[1]
    act_t = jnp.swapaxes(act_seq, 1, 2)

    return pl.pallas_call(
        _update_kernel,
        grid=(B // 2,),
        in_specs=[
            pl.BlockSpec((2, T, D), lambda b: (b, 0, 0)),
            pl.BlockSpec((2, C, T), lambda b: (b, 0, 0)),
            pl.BlockSpec((B, C), lambda b: (0, 0)),
            pl.BlockSpec((C, P, D), lambda b: (0, 0, 0)),
        ],
        out_specs=pl.BlockSpec((C, P, D), lambda b: (0, 0, 0)),
        out_shape=jax.ShapeDtypeStruct((C, P, D), jnp.float32),
        scratch_shapes=[
            pltpu.VMEM((C, D), jnp.float32),
            pltpu.VMEM((C, 128), jnp.float32),
        ],
    )(feats, act_t, vid_label, proto_vectors)
